# baseline (speedup 1.0000x reference)
import functools
import math

import jax
import jax.numpy as jnp
from jax import lax
from jax.experimental import pallas as pl
from jax.experimental.pallas import tpu as pltpu
from jax.experimental.pallas import tpu_sc as plsc

_NC = 2
_NS = 16
_NW = _NC * _NS
_LANES = 16


def _build_sc_lookup(seq, batch, d):
    ppw = seq // _NW
    scale = float(math.sqrt(d))
    mesh = plsc.VectorSubcoreMesh(core_axis_name="c", subcore_axis_name="s")

    @functools.partial(
        pl.kernel,
        mesh=mesh,
        out_type=jax.ShapeDtypeStruct((seq, batch * d), jnp.float32),
        scratch_types=[
            pltpu.VMEM((batch, ppw), jnp.int32),
            pltpu.VMEM((batch, ppw, d), jnp.float32),
            pltpu.VMEM((ppw, d), jnp.float32),
            pltpu.SemaphoreType.DMA,
        ],
    )
    def sc_lookup(table_hbm, xt_hbm, pos_hbm, out_hbm, idxb_v, rows_v, pos_v, sem):
        wid = lax.axis_index("s") * _NC + lax.axis_index("c")
        base = wid * ppw
        pltpu.sync_copy(xt_hbm.at[:, pl.ds(base, ppw)], idxb_v)
        copies = [
            pltpu.async_copy(table_hbm.at[idxb_v.at[b]], rows_v.at[b], sem)
            for b in range(batch)
        ]
        pltpu.sync_copy(pos_hbm.at[pl.ds(base, ppw)], pos_v)
        for cp in copies:
            cp.wait()

        def step(p, carry):
            pos_regs = [pos_v[p, pl.ds(k * _LANES, _LANES)] for k in range(d // _LANES)]
            for b in range(batch):
                for k in range(d // _LANES):
                    sl = pl.ds(k * _LANES, _LANES)
                    rows_v[b, p, sl] = rows_v[b, p, sl] * scale + pos_regs[k]
            return carry

        lax.fori_loop(0, ppw, step, 0)
        for b in range(batch):
            pltpu.sync_copy(
                rows_v.at[b], out_hbm.at[pl.ds(base, ppw), pl.ds(b * d, d)]
            )

    return sc_lookup


def kernel(x, table, pos_emb):
    seq, batch = x.shape
    d = table.shape[1]
    xt = x.T
    pos2 = pos_emb[:seq].reshape(seq, d)
    out = _build_sc_lookup(seq, batch, d)(table, xt, pos2)
    return out.reshape(seq, batch, d)
